# baseline (device time: 11148 ns/iter reference)
import jax
import jax.numpy as jnp
from jax import lax
from jax.experimental import pallas as pl
from jax.experimental.pallas import tpu as pltpu

N_DEV = 4
M = 256
H = M // 2
N_TOT = 1024
CHUNK = N_TOT // N_DEV


def kernel(x):
    x2 = x.reshape(M, N_TOT)

    def body(x_ref, out_ref, recv_a, recv_b, recv_a2, recv_b2, fwd_a, fwd_b,
             send_sems, recv_a_sems, recv_b_sems, recv_a2_sem, recv_b2_sem):

        my = lax.axis_index("i")
        p_y = my ^ 1
        p_x = my ^ 3
        diag = my ^ 2

        barrier_sem = pltpu.get_barrier_semaphore()
        for nbr in [p_y, p_x]:
            pl.semaphore_signal(
                barrier_sem, inc=1,
                device_id=(nbr,), device_id_type=pl.DeviceIdType.MESH,
            )
        pl.semaphore_wait(barrier_sem, 2)

        def rdma(src, dst, ssem, rsem, dev):
            return pltpu.make_async_remote_copy(
                src_ref=src, dst_ref=dst, send_sem=ssem, recv_sem=rsem,
                device_id=(dev,), device_id_type=pl.DeviceIdType.MESH,
            )

        def a_src(c):
            return x_ref.at[pl.ds(0, H), pl.ds(c * CHUNK, CHUNK)]

        def b_src(c):
            return x_ref.at[pl.ds(H, H), pl.ds(c * CHUNK, CHUNK)]

        a1_crit = rdma(a_src(diag), recv_a.at[1],
                       send_sems.at[0], recv_a_sems.at[1], p_y)
        a1_own = rdma(a_src(p_y), recv_a.at[0],
                      send_sems.at[2], recv_a_sems.at[0], p_y)
        b1_crit = rdma(b_src(diag), recv_b.at[1],
                       send_sems.at[1], recv_b_sems.at[1], p_x)
        b1_own = rdma(b_src(p_x), recv_b.at[0],
                      send_sems.at[3], recv_b_sems.at[0], p_x)

        a1_crit.start()
        b1_crit.start()
        a1_own.start()
        b1_own.start()

        a1_crit.wait_recv()
        fwd_a[:, :] = recv_a[1] + x_ref[pl.ds(0, H), pl.ds(p_x * CHUNK, CHUNK)]
        a2 = rdma(fwd_a, recv_a2, send_sems.at[4], recv_a2_sem, p_x)
        a2.start()

        b1_crit.wait_recv()
        fwd_b[:, :] = recv_b[1] + x_ref[pl.ds(H, H), pl.ds(p_y * CHUNK, CHUNK)]
        b2 = rdma(fwd_b, recv_b2, send_sems.at[5], recv_b2_sem, p_y)
        b2.start()

        a1_own.wait_recv()
        a2.wait_recv()
        out_ref[pl.ds(0, H), :] = (
            x_ref[pl.ds(0, H), pl.ds(my * CHUNK, CHUNK)]
            + recv_a[0] + recv_a2[:, :]
        )
        b1_own.wait_recv()
        b2.wait_recv()
        out_ref[pl.ds(H, H), :] = (
            x_ref[pl.ds(H, H), pl.ds(my * CHUNK, CHUNK)]
            + recv_b[0] + recv_b2[:, :]
        )

        for r in (a1_crit, b1_crit, a1_own, b1_own, a2, b2):
            r.wait_send()

    return pl.pallas_call(
        body,
        out_shape=jax.ShapeDtypeStruct((M, CHUNK), jnp.float32),
        in_specs=[pl.BlockSpec(memory_space=pltpu.VMEM)],
        out_specs=pl.BlockSpec(memory_space=pltpu.VMEM),
        scratch_shapes=[
            pltpu.VMEM((2, H, CHUNK), jnp.float32),
            pltpu.VMEM((2, H, CHUNK), jnp.float32),
            pltpu.VMEM((H, CHUNK), jnp.float32),
            pltpu.VMEM((H, CHUNK), jnp.float32),
            pltpu.VMEM((H, CHUNK), jnp.float32),
            pltpu.VMEM((H, CHUNK), jnp.float32),
            pltpu.SemaphoreType.DMA((6,)),
            pltpu.SemaphoreType.DMA((2,)),
            pltpu.SemaphoreType.DMA((2,)),
            pltpu.SemaphoreType.DMA,
            pltpu.SemaphoreType.DMA,
        ],
        compiler_params=pltpu.CompilerParams(collective_id=0),
    )(x2)
